# baseline (speedup 1.0000x reference)
import functools

import jax
import jax.numpy as jnp
from jax import lax
from jax.experimental import pallas as pl
from jax.experimental.pallas import tpu as pltpu
from jax.experimental.pallas import tpu_sc as plsc

_NC = 2
_NS = 16
_NW = _NC * _NS
_L = 16
_ZR = 64
_BATCH = 80


@functools.lru_cache(maxsize=None)
def _build_unpool(N: int, M: int, d: int):
    Mpad = ((M + _L - 1) // _L) * _L
    n_vec_m = Mpad // _L

    assert N % 8 == 0
    units = N // 8
    u_lo = units // _NW
    n_hi = units - _NW * u_lo
    k_lo = u_lo * 8
    k_hi = k_lo + (8 if n_hi else 0)
    nb = (max(k_hi, 1) + _BATCH - 1) // _BATCH
    max_base = (_NW - 1) * k_lo + n_hi * 8
    Npad = ((max_base + nb * _BATCH + _L - 1) // _L) * _L

    mesh = plsc.VectorSubcoreMesh(core_axis_name="c", subcore_axis_name="s")

    @functools.partial(
        pl.kernel,
        out_type=jax.ShapeDtypeStruct((N, d), jnp.float32),
        mesh=mesh,
        compiler_params=pltpu.CompilerParams(needs_layout_passes=False),
        scratch_types=[
            pltpu.VMEM((Mpad,), jnp.int32),
            pltpu.VMEM((Npad,), jnp.int32),
            pltpu.VMEM((_BATCH,), jnp.int32),
            pltpu.VMEM((_BATCH, d), jnp.float32),
            pltpu.SemaphoreType.DMA,
        ],
    )
    def unpool(x_ext_hbm, idx_hbm, out_hbm, idx_v, perm_v, sel_v, rows_v, sem):
        wid = lax.axis_index("s") * _NC + lax.axis_index("c")
        lane = jnp.arange(_L, dtype=jnp.int32)

        if Mpad > M:
            idx_v[pl.ds(Mpad - _L, _L)] = jnp.zeros((_L,), jnp.int32)
        pltpu.sync_copy(idx_hbm, idx_v.at[pl.ds(0, M)])

        neg1 = jnp.full((_L,), -1, jnp.int32)

        def init_body(k, _):
            perm_v[pl.ds(k * _L, _L)] = neg1
            return 0

        lax.fori_loop(0, Npad // _L, init_body, 0, unroll=4)

        def scat_body(k, _):
            base = k * _L
            iv = idx_v[pl.ds(base, _L)]
            vv = base + lane
            valid = vv < M
            for l in range(_L):
                plsc.store_scatter(
                    perm_v, [iv], vv, mask=(lane == l) & valid
                )
            return 0

        lax.fori_loop(0, n_vec_m, scat_body, 0)

        def emit(base2, cnt):
            for g in range(nb):
                gb = base2 + g * _BATCH
                for j in range(_BATCH // _L):
                    rs = gb + j * _L
                    p = perm_v[pl.ds(rs, _L)]
                    nz = M + ((rs + lane) & (_ZR - 1))
                    sel_v[pl.ds(j * _L, _L)] = jnp.where(p >= 0, p, nz)
                pltpu.async_copy(x_ext_hbm.at[sel_v], rows_v, sem).wait()
                sz = min(_BATCH, cnt - g * _BATCH)
                if sz <= 0:
                    continue
                pltpu.sync_copy(
                    rows_v.at[pl.ds(0, sz)], out_hbm.at[pl.ds(gb, sz)]
                )

        @pl.when(wid < n_hi)
        def _():
            emit(wid * k_hi, k_hi)

        @pl.when(wid >= n_hi)
        def _():
            emit(wid * k_lo + n_hi * 8, k_lo)

    return unpool


def kernel(A, X, idx):
    N = A.shape[0]
    M, d = X.shape
    idx32 = idx.astype(jnp.int32)
    x_ext = jnp.concatenate([X, jnp.zeros((_ZR, d), X.dtype)], axis=0)
    new_x = _build_unpool(N, M, d)(x_ext, idx32)
    return (A, new_x)
